# Initial kernel scaffold; baseline (speedup 1.0000x reference)
import jax
import jax.numpy as jnp
from jax.experimental import pallas as pl


def kernel(n_in_feats, r_feats, edge_index, W_O_w, W_O_b, W_I_w, W_I_b, W_S_w, W_S_b, W_R_w, W_R_b):
    raise NotImplementedError("write your pallas kernel here")


# speedup vs baseline: 7.3114x; 7.3114x over previous
import functools

import jax
import jax.numpy as jnp
from jax import lax
from jax.experimental import pallas as pl
from jax.experimental.pallas import tpu as pltpu
from jax.experimental.pallas import tpu_sc as plsc

_NC = 2
_NS = 16
_D = 128
_DA = 144
_CH = 80


def _sc_segment_sums(h_aug, edge_index):
    n = h_aug.shape[0]
    e = edge_index.shape[1]
    half = e // 2
    per_tile = half // _NS
    n_chunks = per_tile // _CH
    rows_per_tile = n // _NS
    assert half % _NS == 0 and per_tile % _CH == 0 and n % _NS == 0

    mesh = plsc.VectorSubcoreMesh(
        core_axis_name="c", subcore_axis_name="s",
        num_cores=_NC, num_subcores=_NS)

    @functools.partial(
        pl.kernel,
        out_type=jax.ShapeDtypeStruct((_NC, n, _DA), jnp.float32),
        mesh=mesh,
        compiler_params=pltpu.CompilerParams(use_tc_tiling_on_sc=False),
        scratch_types=[
            pltpu.VMEM_SHARED((n, _DA), jnp.float32),
            pltpu.VMEM((_CH, _DA), jnp.float32),
            pltpu.VMEM((_CH,), jnp.int32),
            pltpu.VMEM((1, _CH), jnp.int32),
            pltpu.SemaphoreType.DMA,
        ],
    )
    def seg_sum(h_hbm, ei_hbm, zeros_hbm, out_hbm, acc, rows, sidx, didx, sem):
        g = lax.axis_index("c")
        s = lax.axis_index("s")
        row0 = s * rows_per_tile
        pltpu.sync_copy(zeros_hbm, acc.at[pl.ds(row0, rows_per_tile)])
        plsc.subcore_barrier()

        base = g * half + s * per_tile

        def body(i, carry):
            off = base + i * _CH
            pltpu.sync_copy(ei_hbm.at[0, pl.ds(off, _CH)], sidx)
            pltpu.sync_copy(ei_hbm.at[1, pl.ds(off, _CH)], didx.at[0])
            pltpu.async_copy(h_hbm.at[sidx], rows, sem).wait()
            pltpu.sync_copy(rows, acc.at[didx.at[0]], add=True)
            return carry

        lax.fori_loop(0, n_chunks, body, 0)
        plsc.subcore_barrier()
        pltpu.sync_copy(acc.at[pl.ds(row0, rows_per_tile)],
                        out_hbm.at[g, pl.ds(row0, rows_per_tile)])

    zeros = jnp.zeros((rows_per_tile, _DA), jnp.float32)
    return seg_sum(h_aug, edge_index, zeros)


def _tc_dense(h, r2, agg, W_O_w, b_O, W_I_w, b_I, W_S_w, b_S, W_R_w, b_R):
    n = h.shape[0]
    bn = 2000
    grid = n // bn

    def body(h_ref, agg_ref, r_ref, wo_ref, bo_ref, wi_ref, bi_ref,
             ws_ref, bs_ref, wr_ref, br_ref, out_ref, rout_ref):
        i = pl.program_id(0)
        hv = h_ref[...]
        a = agg_ref[...]
        go = a[0, :, :_D]
        do = a[0, :, _D:_D + 1]
        gi = a[1, :, :_D]
        di = a[1, :, _D:_D + 1]
        xs = hv - r_ref[...]
        xo = go - do * hv
        xi = gi - di * hv
        acc = jnp.dot(xs, ws_ref[...], preferred_element_type=jnp.float32)
        acc += jnp.dot(xo, wo_ref[...], preferred_element_type=jnp.float32)
        acc += jnp.dot(xi, wi_ref[...], preferred_element_type=jnp.float32)
        out_ref[...] = acc + bs_ref[...] + do * bo_ref[...] + di * bi_ref[...]

        @pl.when(i == 0)
        def _():
            rout_ref[...] = (
                jnp.dot(r_ref[...], wr_ref[...],
                        preferred_element_type=jnp.float32) + br_ref[...])

    full = lambda shape: pl.BlockSpec(shape, lambda i: tuple(0 for _ in shape))
    return pl.pallas_call(
        body,
        grid=(grid,),
        in_specs=[
            pl.BlockSpec((bn, _D), lambda i: (i, 0)),
            pl.BlockSpec((_NC, bn, _DA), lambda i: (0, i, 0)),
            full((1, _D)),
            full((_D, _D)), full((1, _D)),
            full((_D, _D)), full((1, _D)),
            full((_D, _D)), full((1, _D)),
            full((_D, _D)), full((1, _D)),
        ],
        out_specs=[
            pl.BlockSpec((bn, _D), lambda i: (i, 0)),
            pl.BlockSpec((1, _D), lambda i: (0, 0)),
        ],
        out_shape=[
            jax.ShapeDtypeStruct((n, _D), jnp.float32),
            jax.ShapeDtypeStruct((1, _D), jnp.float32),
        ],
    )(h, agg, r2, W_O_w, b_O, W_I_w, b_I, W_S_w, b_S, W_R_w, b_R)


def kernel(n_in_feats, r_feats, edge_index, W_O_w, W_O_b, W_I_w, W_I_b,
           W_S_w, W_S_b, W_R_w, W_R_b):
    n = n_in_feats.shape[0]
    h_aug = jnp.concatenate(
        [n_in_feats,
         jnp.ones((n, 1), jnp.float32),
         jnp.zeros((n, _DA - _D - 1), jnp.float32)], axis=1)
    agg = _sc_segment_sums(h_aug, edge_index)
    n_out, r_out = _tc_dense(
        n_in_feats, r_feats.reshape(1, _D), agg,
        W_O_w, W_O_b.reshape(1, _D), W_I_w, W_I_b.reshape(1, _D),
        W_S_w, W_S_b.reshape(1, _D), W_R_w, W_R_b.reshape(1, _D))
    return (n_out, r_out.reshape(_D))


# speedup vs baseline: 12.8754x; 1.7610x over previous
import functools

import jax
import jax.numpy as jnp
from jax import lax
from jax.experimental import pallas as pl
from jax.experimental.pallas import tpu as pltpu
from jax.experimental.pallas import tpu_sc as plsc

_NC = 2
_NS = 16
_D = 128
_DA = 144
_CH = 125


def _sc_segment_sums(h_aug, edge_index):
    n = h_aug.shape[0]
    e = edge_index.shape[1]
    half = e // 2
    per_tile = half // _NS
    n_chunks = per_tile // _CH
    n_iter = n_chunks // 2
    rows_per_tile = n // _NS
    assert half % _NS == 0 and per_tile % _CH == 0 and n_chunks % 2 == 0
    assert n % _NS == 0

    ei_t = edge_index.reshape(2, e // _CH, _CH).transpose(1, 0, 2)
    chunks_half = half // _CH

    mesh = plsc.VectorSubcoreMesh(
        core_axis_name="c", subcore_axis_name="s",
        num_cores=_NC, num_subcores=_NS)

    @functools.partial(
        pl.kernel,
        out_type=jax.ShapeDtypeStruct((_NC, n, _DA), jnp.float32),
        mesh=mesh,
        compiler_params=pltpu.CompilerParams(use_tc_tiling_on_sc=False),
        scratch_types=[
            pltpu.VMEM_SHARED((n, _DA), jnp.float32),
            pltpu.VMEM((_CH, _DA), jnp.float32),
            pltpu.VMEM((_CH, _DA), jnp.float32),
            pltpu.VMEM((2, _CH), jnp.int32),
            pltpu.VMEM((2, _CH), jnp.int32),
            pltpu.SemaphoreType.DMA,
            pltpu.SemaphoreType.DMA,
        ],
    )
    def seg_sum(h_hbm, ei_hbm, zeros_hbm, out_hbm, acc, rows0, rows1,
                idx0, idx1, sem0, sem1):
        g = lax.axis_index("c")
        s = lax.axis_index("s")
        row0 = s * rows_per_tile
        pltpu.sync_copy(zeros_hbm, acc.at[pl.ds(row0, rows_per_tile)])
        plsc.subcore_barrier()

        cbase = g * chunks_half + s * n_chunks

        pltpu.sync_copy(ei_hbm.at[cbase], idx0)
        pltpu.async_copy(h_hbm.at[idx0.at[0]], rows0, sem0)

        def body(i, carry):
            c = cbase + 2 * i
            pltpu.sync_copy(ei_hbm.at[c + 1], idx1)
            pltpu.async_copy(h_hbm.at[idx1.at[0]], rows1, sem1)
            pltpu.make_async_copy(h_hbm.at[idx0.at[0]], rows0, sem0).wait()
            pltpu.sync_copy(rows0, acc.at[idx0.at[1]], add=True)

            @pl.when(i < n_iter - 1)
            def _():
                pltpu.sync_copy(ei_hbm.at[c + 2], idx0)
                pltpu.async_copy(h_hbm.at[idx0.at[0]], rows0, sem0)

            pltpu.make_async_copy(h_hbm.at[idx1.at[0]], rows1, sem1).wait()
            pltpu.sync_copy(rows1, acc.at[idx1.at[1]], add=True)
            return carry

        lax.fori_loop(0, n_iter, body, 0)
        plsc.subcore_barrier()
        pltpu.sync_copy(acc.at[pl.ds(row0, rows_per_tile)],
                        out_hbm.at[g, pl.ds(row0, rows_per_tile)])

    zeros = jnp.zeros((rows_per_tile, _DA), jnp.float32)
    return seg_sum(h_aug, ei_t, zeros)


def _tc_dense(h, r2, agg, W_O_w, b_O, W_I_w, b_I, W_S_w, b_S, W_R_w, b_R):
    n = h.shape[0]
    bn = 2000
    grid = n // bn

    def body(h_ref, agg_ref, r_ref, wo_ref, bo_ref, wi_ref, bi_ref,
             ws_ref, bs_ref, wr_ref, br_ref, out_ref, rout_ref):
        i = pl.program_id(0)
        hv = h_ref[...]
        a = agg_ref[...]
        go = a[0, :, :_D]
        do = a[0, :, _D:_D + 1]
        gi = a[1, :, :_D]
        di = a[1, :, _D:_D + 1]
        xs = hv - r_ref[...]
        xo = go - do * hv
        xi = gi - di * hv
        hi = jax.lax.Precision.HIGHEST
        acc = jnp.dot(xs, ws_ref[...], precision=hi,
                      preferred_element_type=jnp.float32)
        acc += jnp.dot(xo, wo_ref[...], precision=hi,
                       preferred_element_type=jnp.float32)
        acc += jnp.dot(xi, wi_ref[...], precision=hi,
                       preferred_element_type=jnp.float32)
        out_ref[...] = acc + bs_ref[...] + do * bo_ref[...] + di * bi_ref[...]

        @pl.when(i == 0)
        def _():
            rout_ref[...] = (
                jnp.dot(r_ref[...], wr_ref[...],
                        precision=jax.lax.Precision.HIGHEST,
                        preferred_element_type=jnp.float32) + br_ref[...])

    full = lambda shape: pl.BlockSpec(shape, lambda i: tuple(0 for _ in shape))
    return pl.pallas_call(
        body,
        grid=(grid,),
        in_specs=[
            pl.BlockSpec((bn, _D), lambda i: (i, 0)),
            pl.BlockSpec((_NC, bn, _DA), lambda i: (0, i, 0)),
            full((1, _D)),
            full((_D, _D)), full((1, _D)),
            full((_D, _D)), full((1, _D)),
            full((_D, _D)), full((1, _D)),
            full((_D, _D)), full((1, _D)),
        ],
        out_specs=[
            pl.BlockSpec((bn, _D), lambda i: (i, 0)),
            pl.BlockSpec((1, _D), lambda i: (0, 0)),
        ],
        out_shape=[
            jax.ShapeDtypeStruct((n, _D), jnp.float32),
            jax.ShapeDtypeStruct((1, _D), jnp.float32),
        ],
    )(h, agg, r2, W_O_w, b_O, W_I_w, b_I, W_S_w, b_S, W_R_w, b_R)


def kernel(n_in_feats, r_feats, edge_index, W_O_w, W_O_b, W_I_w, W_I_b,
           W_S_w, W_S_b, W_R_w, W_R_b):
    n = n_in_feats.shape[0]
    h_aug = jnp.concatenate(
        [n_in_feats,
         jnp.ones((n, 1), jnp.float32),
         jnp.zeros((n, _DA - _D - 1), jnp.float32)], axis=1)
    agg = _sc_segment_sums(h_aug, edge_index)
    n_out, r_out = _tc_dense(
        n_in_feats, r_feats.reshape(1, _D), agg,
        W_O_w, W_O_b.reshape(1, _D), W_I_w, W_I_b.reshape(1, _D),
        W_S_w, W_S_b.reshape(1, _D), W_R_w, W_R_b.reshape(1, _D))
    return (n_out, r_out.reshape(_D))


# speedup vs baseline: 14.3227x; 1.1124x over previous
import functools

import jax
import jax.numpy as jnp
from jax import lax
from jax.experimental import pallas as pl
from jax.experimental.pallas import tpu as pltpu
from jax.experimental.pallas import tpu_sc as plsc

_NC = 2
_NS = 16
_D = 128
_CH = 125
_NP = 10240


def _sc_segment_sums(h, edge_index):
    e = edge_index.shape[1]
    half = e // 2
    per_tile = half // _NS
    n_chunks = per_tile // _CH
    n_iter = n_chunks // 2
    rows_per_tile = _NP // _NS
    assert half % _NS == 0 and per_tile % _CH == 0 and n_chunks % 2 == 0

    ei3 = edge_index.reshape(2, e // _CH, _CH)
    chunks_half = half // _CH

    mesh = plsc.VectorSubcoreMesh(
        core_axis_name="c", subcore_axis_name="s",
        num_cores=_NC, num_subcores=_NS)

    @functools.partial(
        pl.kernel,
        out_type=(jax.ShapeDtypeStruct((_NC, _NP, _D), jnp.float32),
                  jax.ShapeDtypeStruct((_NC, _NP), jnp.float32)),
        mesh=mesh,
        compiler_params=pltpu.CompilerParams(use_tc_tiling_on_sc=False),
        scratch_types=[
            pltpu.VMEM_SHARED((_NP, _D), jnp.float32),
            pltpu.VMEM_SHARED((_NP,), jnp.float32),
            pltpu.VMEM((_CH, _D), jnp.float32),
            pltpu.VMEM((_CH, _D), jnp.float32),
            pltpu.VMEM((1, _CH), jnp.int32),
            pltpu.VMEM((1, _CH), jnp.int32),
            pltpu.VMEM((1, _CH), jnp.int32),
            pltpu.VMEM((1, _CH), jnp.int32),
            pltpu.VMEM((128,), jnp.float32),
            pltpu.SemaphoreType.DMA,
            pltpu.SemaphoreType.DMA,
        ],
    )
    def seg_sum(h_hbm, ei_hbm, zf_hbm, zd_hbm, feats_hbm, deg_hbm,
                acc, dacc, rows0, rows1, s0, d0, s1, d1, ones, sem0, sem1):
        g = lax.axis_index("c")
        s = lax.axis_index("s")
        row0 = s * rows_per_tile
        pltpu.sync_copy(zf_hbm, acc.at[pl.ds(row0, rows_per_tile)])
        pltpu.sync_copy(zd_hbm, dacc.at[pl.ds(row0, rows_per_tile)])
        for j in range(8):
            ones[pl.ds(16 * j, 16)] = jnp.ones((16,), jnp.float32)
        plsc.subcore_barrier()

        cb = g * chunks_half + s * n_chunks

        pltpu.sync_copy(ei_hbm.at[0, cb], s0.at[0])
        pltpu.sync_copy(ei_hbm.at[1, cb], d0.at[0])
        pltpu.async_copy(h_hbm.at[s0.at[0]], rows0, sem0)

        def body(i, carry):
            c = cb + 2 * i
            pltpu.sync_copy(ei_hbm.at[0, c + 1], s1.at[0])
            pltpu.sync_copy(ei_hbm.at[1, c + 1], d1.at[0])
            pltpu.async_copy(h_hbm.at[s1.at[0]], rows1, sem1)
            pltpu.make_async_copy(h_hbm.at[s0.at[0]], rows0, sem0).wait()
            pltpu.sync_copy(rows0, acc.at[d0.at[0]], add=True)
            pltpu.sync_copy(ones.at[pl.ds(0, _CH)], dacc.at[d0.at[0]],
                            add=True)

            @pl.when(i < n_iter - 1)
            def _():
                pltpu.sync_copy(ei_hbm.at[0, c + 2], s0.at[0])
                pltpu.sync_copy(ei_hbm.at[1, c + 2], d0.at[0])
                pltpu.async_copy(h_hbm.at[s0.at[0]], rows0, sem0)

            pltpu.make_async_copy(h_hbm.at[s1.at[0]], rows1, sem1).wait()
            pltpu.sync_copy(rows1, acc.at[d1.at[0]], add=True)
            pltpu.sync_copy(ones.at[pl.ds(0, _CH)], dacc.at[d1.at[0]],
                            add=True)
            return carry

        lax.fori_loop(0, n_iter, body, 0)
        plsc.subcore_barrier()
        pltpu.sync_copy(acc.at[pl.ds(row0, rows_per_tile)],
                        feats_hbm.at[g, pl.ds(row0, rows_per_tile)])
        pltpu.sync_copy(dacc.at[pl.ds(row0, rows_per_tile)],
                        deg_hbm.at[g, pl.ds(row0, rows_per_tile)])

    zf = jnp.zeros((rows_per_tile, _D), jnp.float32)
    zd = jnp.zeros((rows_per_tile,), jnp.float32)
    return seg_sum(h, ei3, zf, zd)


def _tc_dense(h, r2, feats, deg, W_O_w, b_O, W_I_w, b_I, W_S_w, b_S,
              W_R_w, b_R):
    n = h.shape[0]
    bn = 2048
    grid = _NP // bn
    sub = bn // _D

    def body(h_ref, f_ref, deg_ref, r_ref, wo_ref, bo_ref, wi_ref, bi_ref,
             ws_ref, bs_ref, wr_ref, br_ref, out_ref, rout_ref):
        i = pl.program_id(0)
        hv = h_ref[...]
        h3 = hv.reshape(sub, _D, _D)
        d3 = deg_ref[...].reshape(_NC, sub, _D)
        do = d3[0][:, :, None]
        di = d3[1][:, :, None]
        xs = hv - r_ref[...]
        xo = (f_ref[0].reshape(sub, _D, _D) - do * h3).reshape(bn, _D)
        xi = (f_ref[1].reshape(sub, _D, _D) - di * h3).reshape(bn, _D)
        hi = jax.lax.Precision.HIGHEST
        acc = jnp.dot(xs, ws_ref[...], precision=hi,
                      preferred_element_type=jnp.float32)
        acc += jnp.dot(xo, wo_ref[...], precision=hi,
                       preferred_element_type=jnp.float32)
        acc += jnp.dot(xi, wi_ref[...], precision=hi,
                       preferred_element_type=jnp.float32)
        bias = (do * bo_ref[...].reshape(1, 1, _D)
                + di * bi_ref[...].reshape(1, 1, _D)).reshape(bn, _D)
        out_ref[...] = acc + bs_ref[...] + bias

        @pl.when(i == 0)
        def _():
            rout_ref[...] = (
                jnp.dot(r_ref[...], wr_ref[...],
                        precision=jax.lax.Precision.HIGHEST,
                        preferred_element_type=jnp.float32) + br_ref[...])

    full = lambda shape: pl.BlockSpec(shape, lambda i: tuple(0 for _ in shape))
    return pl.pallas_call(
        body,
        grid=(grid,),
        in_specs=[
            pl.BlockSpec((bn, _D), lambda i: (i, 0)),
            pl.BlockSpec((_NC, bn, _D), lambda i: (0, i, 0)),
            pl.BlockSpec((_NC, bn), lambda i: (0, i)),
            full((1, _D)),
            full((_D, _D)), full((1, _D)),
            full((_D, _D)), full((1, _D)),
            full((_D, _D)), full((1, _D)),
            full((_D, _D)), full((1, _D)),
        ],
        out_specs=[
            pl.BlockSpec((bn, _D), lambda i: (i, 0)),
            pl.BlockSpec((1, _D), lambda i: (0, 0)),
        ],
        out_shape=[
            jax.ShapeDtypeStruct((n, _D), jnp.float32),
            jax.ShapeDtypeStruct((1, _D), jnp.float32),
        ],
    )(h, feats, deg, r2, W_O_w, b_O, W_I_w, b_I, W_S_w, b_S, W_R_w, b_R)


def kernel(n_in_feats, r_feats, edge_index, W_O_w, W_O_b, W_I_w, W_I_b,
           W_S_w, W_S_b, W_R_w, W_R_b):
    feats, deg = _sc_segment_sums(n_in_feats, edge_index)
    n_out, r_out = _tc_dense(
        n_in_feats, r_feats.reshape(1, _D), feats, deg,
        W_O_w, W_O_b.reshape(1, _D), W_I_w, W_I_b.reshape(1, _D),
        W_S_w, W_S_b.reshape(1, _D), W_R_w, W_R_b.reshape(1, _D))
    return (n_out, r_out.reshape(_D))


# speedup vs baseline: 14.8137x; 1.0343x over previous
import functools

import jax
import jax.numpy as jnp
from jax import lax
from jax.experimental import pallas as pl
from jax.experimental.pallas import tpu as pltpu
from jax.experimental.pallas import tpu_sc as plsc

_NC = 2
_NS = 16
_D = 128
_CH = 128
_NP = 10240


def _sc_segment_sums(h, edge_index):
    n = h.shape[0]
    e = edge_index.shape[1]
    half = e // 2
    chunks_half = half // _CH
    base_chunks = chunks_half // _NS
    extra = chunks_half - base_chunks * _NS
    n_pairs = base_chunks // 2
    rows_per_tile = n // _NS
    drows_per_tile = _NP // _NS
    assert half % _CH == 0 and base_chunks % 2 == 0 and n % _NS == 0
    assert extra < _NS

    ei = edge_index.reshape(e // 64, _CH)
    dplane = e // _CH

    mesh = plsc.VectorSubcoreMesh(
        core_axis_name="c", subcore_axis_name="s",
        num_cores=_NC, num_subcores=_NS)

    @functools.partial(
        pl.kernel,
        out_type=(jax.ShapeDtypeStruct((_NC, n, _D), jnp.float32),
                  jax.ShapeDtypeStruct((_NC, _NP), jnp.float32)),
        mesh=mesh,
        compiler_params=pltpu.CompilerParams(use_tc_tiling_on_sc=False),
        scratch_types=[
            pltpu.VMEM_SHARED((n, _D), jnp.float32),
            pltpu.VMEM_SHARED((_NP,), jnp.float32),
            pltpu.VMEM((_CH, _D), jnp.float32),
            pltpu.VMEM((_CH, _D), jnp.float32),
            pltpu.VMEM((1, _CH), jnp.int32),
            pltpu.VMEM((1, _CH), jnp.int32),
            pltpu.VMEM((1, _CH), jnp.int32),
            pltpu.VMEM((1, _CH), jnp.int32),
            pltpu.VMEM((_CH,), jnp.float32),
            pltpu.SemaphoreType.DMA,
            pltpu.SemaphoreType.DMA,
            pltpu.SemaphoreType.DMA,
            pltpu.SemaphoreType.DMA,
        ],
    )
    def seg_sum(h_hbm, ei_hbm, zf_hbm, zd_hbm, feats_hbm, deg_hbm,
                acc, dacc, rows0, rows1, s0, d0, s1, d1, ones,
                sem0, sem1, semo0, semo1):
        g = lax.axis_index("c")
        s = lax.axis_index("s")
        frow = s * rows_per_tile
        drow = s * drows_per_tile
        pltpu.sync_copy(zf_hbm, acc.at[pl.ds(frow, rows_per_tile)])
        pltpu.sync_copy(zd_hbm, dacc.at[pl.ds(drow, drows_per_tile)])
        for j in range(8):
            ones[pl.ds(16 * j, 16)] = jnp.ones((16,), jnp.float32)
        plsc.subcore_barrier()

        cb = g * chunks_half + s * base_chunks + jnp.minimum(s, extra)

        def ones_wait(semo):
            pltpu.make_async_copy(
                ones, dacc.at[d0.at[0]], semo).wait()

        pltpu.sync_copy(ei_hbm.at[cb], s0.at[0])
        pltpu.sync_copy(ei_hbm.at[dplane + cb], d0.at[0])
        pltpu.async_copy(h_hbm.at[s0.at[0]], rows0, sem0)
        pltpu.async_copy(ones, dacc.at[d0.at[0]], semo0, add=True)

        def body(i, carry):
            c = cb + 2 * i

            @pl.when(i > 0)
            def _():
                ones_wait(semo1)
            pltpu.sync_copy(ei_hbm.at[c + 1], s1.at[0])
            pltpu.sync_copy(ei_hbm.at[dplane + c + 1], d1.at[0])
            pltpu.async_copy(h_hbm.at[s1.at[0]], rows1, sem1)
            pltpu.async_copy(ones, dacc.at[d1.at[0]], semo1, add=True)
            pltpu.make_async_copy(h_hbm.at[s0.at[0]], rows0, sem0).wait()
            pltpu.sync_copy(rows0, acc.at[d0.at[0]], add=True)

            @pl.when(i < n_pairs - 1)
            def _():
                ones_wait(semo0)
                pltpu.sync_copy(ei_hbm.at[c + 2], s0.at[0])
                pltpu.sync_copy(ei_hbm.at[dplane + c + 2], d0.at[0])
                pltpu.async_copy(h_hbm.at[s0.at[0]], rows0, sem0)
                pltpu.async_copy(ones, dacc.at[d0.at[0]], semo0, add=True)

            pltpu.make_async_copy(h_hbm.at[s1.at[0]], rows1, sem1).wait()
            pltpu.sync_copy(rows1, acc.at[d1.at[0]], add=True)
            return carry

        lax.fori_loop(0, n_pairs, body, 0)

        @pl.when(s < extra)
        def _():
            c = cb + base_chunks
            ones_wait(semo0)
            pltpu.sync_copy(ei_hbm.at[c], s0.at[0])
            pltpu.sync_copy(ei_hbm.at[dplane + c], d0.at[0])
            pltpu.async_copy(h_hbm.at[s0.at[0]], rows0, sem0)
            pltpu.async_copy(ones, dacc.at[d0.at[0]], semo0, add=True)
            pltpu.make_async_copy(h_hbm.at[s0.at[0]], rows0, sem0).wait()
            pltpu.sync_copy(rows0, acc.at[d0.at[0]], add=True)

        ones_wait(semo0)
        ones_wait(semo1)
        plsc.subcore_barrier()
        pltpu.sync_copy(acc.at[pl.ds(frow, rows_per_tile)],
                        feats_hbm.at[g, pl.ds(frow, rows_per_tile)])
        pltpu.sync_copy(dacc.at[pl.ds(drow, drows_per_tile)],
                        deg_hbm.at[g, pl.ds(drow, drows_per_tile)])

    zf = jnp.zeros((rows_per_tile, _D), jnp.float32)
    zd = jnp.zeros((drows_per_tile,), jnp.float32)
    return seg_sum(h, ei, zf, zd)


def _tc_dense(h, r2, feats, deg, W_O_w, b_O, W_I_w, b_I, W_S_w, b_S,
              W_R_w, b_R):
    n = h.shape[0]
    bn = 2048
    grid = _NP // bn
    sub = bn // _D

    def body(h_ref, f_ref, deg_ref, r_ref, wo_ref, bo_ref, wi_ref, bi_ref,
             ws_ref, bs_ref, wr_ref, br_ref, out_ref, rout_ref):
        i = pl.program_id(0)
        hv = h_ref[...]
        h3 = hv.reshape(sub, _D, _D)
        d3 = deg_ref[...].reshape(_NC, sub, _D)
        do = d3[0][:, :, None]
        di = d3[1][:, :, None]
        xs = hv - r_ref[...]
        xo = (f_ref[0].reshape(sub, _D, _D) - do * h3).reshape(bn, _D)
        xi = (f_ref[1].reshape(sub, _D, _D) - di * h3).reshape(bn, _D)
        hi = jax.lax.Precision.HIGHEST
        acc = jnp.dot(xs, ws_ref[...], precision=hi,
                      preferred_element_type=jnp.float32)
        acc += jnp.dot(xo, wo_ref[...], precision=hi,
                       preferred_element_type=jnp.float32)
        acc += jnp.dot(xi, wi_ref[...], precision=hi,
                       preferred_element_type=jnp.float32)
        bias = (do * bo_ref[...].reshape(1, 1, _D)
                + di * bi_ref[...].reshape(1, 1, _D)).reshape(bn, _D)
        out_ref[...] = acc + bs_ref[...] + bias

        @pl.when(i == 0)
        def _():
            rout_ref[...] = (
                jnp.dot(r_ref[...], wr_ref[...],
                        precision=jax.lax.Precision.HIGHEST,
                        preferred_element_type=jnp.float32) + br_ref[...])

    full = lambda shape: pl.BlockSpec(shape, lambda i: tuple(0 for _ in shape))
    return pl.pallas_call(
        body,
        grid=(grid,),
        in_specs=[
            pl.BlockSpec((bn, _D), lambda i: (i, 0)),
            pl.BlockSpec((_NC, bn, _D), lambda i: (0, i, 0)),
            pl.BlockSpec((_NC, bn), lambda i: (0, i)),
            full((1, _D)),
            full((_D, _D)), full((1, _D)),
            full((_D, _D)), full((1, _D)),
            full((_D, _D)), full((1, _D)),
            full((_D, _D)), full((1, _D)),
        ],
        out_specs=[
            pl.BlockSpec((bn, _D), lambda i: (i, 0)),
            pl.BlockSpec((1, _D), lambda i: (0, 0)),
        ],
        out_shape=[
            jax.ShapeDtypeStruct((n, _D), jnp.float32),
            jax.ShapeDtypeStruct((1, _D), jnp.float32),
        ],
    )(h, feats, deg, r2, W_O_w, b_O, W_I_w, b_I, W_S_w, b_S, W_R_w, b_R)


def kernel(n_in_feats, r_feats, edge_index, W_O_w, W_O_b, W_I_w, W_I_b,
           W_S_w, W_S_b, W_R_w, W_R_b):
    feats, deg = _sc_segment_sums(n_in_feats, edge_index)
    n_out, r_out = _tc_dense(
        n_in_feats, r_feats.reshape(1, _D), feats, deg,
        W_O_w, W_O_b.reshape(1, _D), W_I_w, W_I_b.reshape(1, _D),
        W_S_w, W_S_b.reshape(1, _D), W_R_w, W_R_b.reshape(1, _D))
    return (n_out, r_out.reshape(_D))


# speedup vs baseline: 17.3320x; 1.1700x over previous
import functools

import jax
import jax.numpy as jnp
from jax import lax
from jax.experimental import pallas as pl
from jax.experimental.pallas import tpu as pltpu
from jax.experimental.pallas import tpu_sc as plsc

_NC = 2
_NS = 16
_D = 128
_CH = 128
_NP = 10240


def _sc_segment_sums(h, edge_index):
    n = h.shape[0]
    e = edge_index.shape[1]
    half = e // 2
    chunks_half = half // _CH
    base_chunks = chunks_half // _NS
    extra = chunks_half - base_chunks * _NS
    n_pairs = base_chunks // 2
    rows_per_tile = n // _NS
    drows_per_tile = _NP // _NS
    assert half % _CH == 0 and base_chunks % 2 == 0 and n % _NS == 0
    assert extra < _NS

    ei = edge_index.reshape(e // 64, _CH)
    dplane = e // _CH

    mesh = plsc.VectorSubcoreMesh(
        core_axis_name="c", subcore_axis_name="s",
        num_cores=_NC, num_subcores=_NS)

    @functools.partial(
        pl.kernel,
        out_type=(jax.ShapeDtypeStruct((_NC, n, _D), jnp.float32),
                  jax.ShapeDtypeStruct((_NC, _NP), jnp.float32)),
        mesh=mesh,
        compiler_params=pltpu.CompilerParams(use_tc_tiling_on_sc=False),
        scratch_types=[
            pltpu.VMEM_SHARED((n, _D), jnp.float32),
            pltpu.VMEM_SHARED((_NP,), jnp.float32),
            pltpu.VMEM((_CH, _D), jnp.float32),
            pltpu.VMEM((_CH, _D), jnp.float32),
            pltpu.VMEM((1, _CH), jnp.int32),
            pltpu.VMEM((1, _CH), jnp.int32),
            pltpu.VMEM((base_chunks + 1, _CH), jnp.int32),
            pltpu.VMEM((_CH,), jnp.float32),
            pltpu.SemaphoreType.DMA,
            pltpu.SemaphoreType.DMA,
            pltpu.SemaphoreType.DMA,
        ],
    )
    def seg_sum(h_hbm, ei_hbm, zf_hbm, zd_hbm, feats_hbm, deg_hbm,
                acc, dacc, rows0, rows1, s0, s1, dall, ones,
                sem0, sem1, semo):
        g = lax.axis_index("c")
        s = lax.axis_index("s")
        frow = s * rows_per_tile
        drow = s * drows_per_tile
        pltpu.sync_copy(zf_hbm, acc.at[pl.ds(frow, rows_per_tile)])
        pltpu.sync_copy(zd_hbm, dacc.at[pl.ds(drow, drows_per_tile)])
        for j in range(8):
            ones[pl.ds(16 * j, 16)] = jnp.ones((16,), jnp.float32)

        cb = g * chunks_half + s * base_chunks + jnp.minimum(s, extra)
        pltpu.sync_copy(ei_hbm.at[pl.ds(dplane + cb, base_chunks)],
                        dall.at[pl.ds(0, base_chunks)])

        @pl.when(s < extra)
        def _():
            pltpu.sync_copy(ei_hbm.at[dplane + cb + base_chunks],
                            dall.at[base_chunks])
        plsc.subcore_barrier()

        def ones_go(k):
            pltpu.async_copy(ones, dacc.at[dall.at[k]], semo, add=True)

        def ones_wait(_i, carry):
            pltpu.make_async_copy(ones, dacc.at[dall.at[0]], semo).wait()
            return carry

        pltpu.sync_copy(ei_hbm.at[cb], s0.at[0])
        pltpu.async_copy(h_hbm.at[s0.at[0]], rows0, sem0)
        ones_go(0)

        def body(i, carry):
            c = cb + 2 * i
            k = 2 * i
            pltpu.sync_copy(ei_hbm.at[c + 1], s1.at[0])
            pltpu.async_copy(h_hbm.at[s1.at[0]], rows1, sem1)
            ones_go(k + 1)
            pltpu.make_async_copy(h_hbm.at[s0.at[0]], rows0, sem0).wait()
            pltpu.sync_copy(rows0, acc.at[dall.at[k]], add=True)

            @pl.when(i < n_pairs - 1)
            def _():
                pltpu.sync_copy(ei_hbm.at[c + 2], s0.at[0])
                pltpu.async_copy(h_hbm.at[s0.at[0]], rows0, sem0)
                ones_go(k + 2)

            pltpu.make_async_copy(h_hbm.at[s1.at[0]], rows1, sem1).wait()
            pltpu.sync_copy(rows1, acc.at[dall.at[k + 1]], add=True)
            return carry

        lax.fori_loop(0, n_pairs, body, 0)

        @pl.when(s < extra)
        def _():
            c = cb + base_chunks
            pltpu.sync_copy(ei_hbm.at[c], s0.at[0])
            pltpu.async_copy(h_hbm.at[s0.at[0]], rows0, sem0)
            ones_go(base_chunks)
            pltpu.make_async_copy(h_hbm.at[s0.at[0]], rows0, sem0).wait()
            pltpu.sync_copy(rows0, acc.at[dall.at[base_chunks]], add=True)

        lax.fori_loop(0, base_chunks + jnp.where(s < extra, 1, 0),
                      ones_wait, 0)
        plsc.subcore_barrier()
        pltpu.sync_copy(acc.at[pl.ds(frow, rows_per_tile)],
                        feats_hbm.at[g, pl.ds(frow, rows_per_tile)])
        pltpu.sync_copy(dacc.at[pl.ds(drow, drows_per_tile)],
                        deg_hbm.at[g, pl.ds(drow, drows_per_tile)])

    zf = jnp.zeros((rows_per_tile, _D), jnp.float32)
    zd = jnp.zeros((drows_per_tile,), jnp.float32)
    return seg_sum(h, ei, zf, zd)


def _tc_dense(h, r2, feats, deg, W_O_w, b_O, W_I_w, b_I, W_S_w, b_S,
              W_R_w, b_R):
    n = h.shape[0]
    bn = 2048
    grid = _NP // bn
    sub = bn // _D

    def body(h_ref, f_ref, deg_ref, r_ref, wo_ref, bo_ref, wi_ref, bi_ref,
             ws_ref, bs_ref, wr_ref, br_ref, out_ref, rout_ref):
        i = pl.program_id(0)
        hv = h_ref[...]
        h3 = hv.reshape(sub, _D, _D)
        d3 = deg_ref[...].reshape(_NC, sub, _D)
        do = d3[0][:, :, None]
        di = d3[1][:, :, None]
        xs = hv - r_ref[...]
        xo = (f_ref[0].reshape(sub, _D, _D) - do * h3).reshape(bn, _D)
        xi = (f_ref[1].reshape(sub, _D, _D) - di * h3).reshape(bn, _D)
        hi = jax.lax.Precision.HIGHEST
        acc = jnp.dot(xs, ws_ref[...], precision=hi,
                      preferred_element_type=jnp.float32)
        acc += jnp.dot(xo, wo_ref[...], precision=hi,
                       preferred_element_type=jnp.float32)
        acc += jnp.dot(xi, wi_ref[...], precision=hi,
                       preferred_element_type=jnp.float32)
        bias = (do * bo_ref[...].reshape(1, 1, _D)
                + di * bi_ref[...].reshape(1, 1, _D)).reshape(bn, _D)
        out_ref[...] = acc + bs_ref[...] + bias

        @pl.when(i == 0)
        def _():
            rout_ref[...] = (
                jnp.dot(r_ref[...], wr_ref[...],
                        precision=jax.lax.Precision.HIGHEST,
                        preferred_element_type=jnp.float32) + br_ref[...])

    full = lambda shape: pl.BlockSpec(shape, lambda i: tuple(0 for _ in shape))
    return pl.pallas_call(
        body,
        grid=(grid,),
        in_specs=[
            pl.BlockSpec((bn, _D), lambda i: (i, 0)),
            pl.BlockSpec((_NC, bn, _D), lambda i: (0, i, 0)),
            pl.BlockSpec((_NC, bn), lambda i: (0, i)),
            full((1, _D)),
            full((_D, _D)), full((1, _D)),
            full((_D, _D)), full((1, _D)),
            full((_D, _D)), full((1, _D)),
            full((_D, _D)), full((1, _D)),
        ],
        out_specs=[
            pl.BlockSpec((bn, _D), lambda i: (i, 0)),
            pl.BlockSpec((1, _D), lambda i: (0, 0)),
        ],
        out_shape=[
            jax.ShapeDtypeStruct((n, _D), jnp.float32),
            jax.ShapeDtypeStruct((1, _D), jnp.float32),
        ],
    )(h, feats, deg, r2, W_O_w, b_O, W_I_w, b_I, W_S_w, b_S, W_R_w, b_R)


def kernel(n_in_feats, r_feats, edge_index, W_O_w, W_O_b, W_I_w, W_I_b,
           W_S_w, W_S_b, W_R_w, W_R_b):
    feats, deg = _sc_segment_sums(n_in_feats, edge_index)
    n_out, r_out = _tc_dense(
        n_in_feats, r_feats.reshape(1, _D), feats, deg,
        W_O_w, W_O_b.reshape(1, _D), W_I_w, W_I_b.reshape(1, _D),
        W_S_w, W_S_b.reshape(1, _D), W_R_w, W_R_b.reshape(1, _D))
    return (n_out, r_out.reshape(_D))
